# baseline (speedup 1.0000x reference)
import functools

import jax
import jax.numpy as jnp
from jax import lax
from jax.experimental import pallas as pl
from jax.experimental.pallas import tpu as pltpu
from jax.experimental.pallas import tpu_sc as plsc

B = 16384
D = 128
H1 = 256
H2 = 512

NC = 2
NS = 16
NW = NC * NS
B_PER_W = B // NW
CH = 128
K = B_PER_W // CH

_sc_mesh = plsc.VectorSubcoreMesh(core_axis_name="c", subcore_axis_name="s")


@functools.partial(
    pl.kernel,
    out_type=jax.ShapeDtypeStruct((B, D), jnp.float32),
    mesh=_sc_mesh,
    scratch_types=[
        pltpu.VMEM((K, CH), jnp.int32),
        pltpu.VMEM((K, CH, D), jnp.float32),
        pltpu.SemaphoreType.DMA,
    ],
)
def _sc_gather(table_hbm, idx_hbm, out_hbm, idx_v, rows_v, sem):
    wid = lax.axis_index("s") * NC + lax.axis_index("c")
    pltpu.sync_copy(idx_hbm.at[wid], idx_v)
    copies = [
        pltpu.async_copy(table_hbm.at[idx_v.at[j]], rows_v.at[j], sem)
        for j in range(K)
    ]
    for j in range(K):
        copies[j].wait()
        pltpu.sync_copy(rows_v.at[j], out_hbm.at[pl.ds(wid * B_PER_W + j * CH, CH)])


def _mlp_body(emb_ref, w1_ref, b1_ref, w2_ref, b2_ref, out_ref):
    dn = (((1,), (1,)), ((), ()))
    h = lax.dot_general(emb_ref[...], w1_ref[...], dn,
                        preferred_element_type=jnp.float32)
    h = h + b1_ref[...]
    h = h * jax.nn.sigmoid(h)
    out_ref[...] = lax.dot_general(h, w2_ref[...], dn,
                                   preferred_element_type=jnp.float32) + b2_ref[...]


BM = 1024


def _mlp(emb, w1t, b1, w2t, b2):
    return pl.pallas_call(
        _mlp_body,
        grid=(B // BM,),
        in_specs=[
            pl.BlockSpec((BM, D), lambda i: (i, 0)),
            pl.BlockSpec((H1, D), lambda i: (0, 0)),
            pl.BlockSpec((H1,), lambda i: (0,)),
            pl.BlockSpec((H2, H1), lambda i: (0, 0)),
            pl.BlockSpec((H2,), lambda i: (0,)),
        ],
        out_specs=pl.BlockSpec((BM, H2), lambda i: (i, 0)),
        out_shape=jax.ShapeDtypeStruct((B, H2), jnp.float32),
    )(emb, w1t, b1, w2t, b2)


def kernel(class_labels, emb_table, W1, b1, W2, b2):
    idx = class_labels.astype(jnp.int32).reshape(NW, K, CH)
    emb = _sc_gather(emb_table, idx)
    return _mlp(emb, W1, b1, W2, b2)


# speedup vs baseline: 1.1075x; 1.1075x over previous
import functools

import jax
import jax.numpy as jnp
from jax import lax
from jax.experimental import pallas as pl
from jax.experimental.pallas import tpu as pltpu
from jax.experimental.pallas import tpu_sc as plsc

B = 16384
D = 128
H1 = 256
H2 = 512

NC = 2
NS = 16
NW = NC * NS
B_PER_W = B // NW
CH = 128
K = B_PER_W // CH

_sc_mesh = plsc.VectorSubcoreMesh(core_axis_name="c", subcore_axis_name="s")


@functools.partial(
    pl.kernel,
    out_type=jax.ShapeDtypeStruct((B, D), jnp.float32),
    mesh=_sc_mesh,
    scratch_types=[
        pltpu.VMEM((K, CH), jnp.int32),
        pltpu.VMEM((K, CH, D), jnp.float32),
        pltpu.SemaphoreType.DMA,
    ],
)
def _sc_gather(table_hbm, idx_hbm, out_hbm, idx_v, rows_v, sem):
    wid = lax.axis_index("s") * NC + lax.axis_index("c")
    pltpu.sync_copy(idx_hbm.at[wid], idx_v)
    copies = [
        pltpu.async_copy(table_hbm.at[idx_v.at[j]], rows_v.at[j], sem)
        for j in range(K)
    ]
    for j in range(K):
        copies[j].wait()
        pltpu.sync_copy(rows_v.at[j], out_hbm.at[pl.ds(wid * B_PER_W + j * CH, CH)])


def _mlp_body(emb_ref, w1_ref, b1_ref, w2_ref, b2_ref, out_ref):
    dn = (((1,), (1,)), ((), ()))
    h = lax.dot_general(emb_ref[...], w1_ref[...], dn,
                        preferred_element_type=jnp.float32)
    h = h + b1_ref[...]
    h = h * jax.nn.sigmoid(h)
    out_ref[...] = lax.dot_general(h, w2_ref[...], dn,
                                   preferred_element_type=jnp.float32) + b2_ref[...]


BM = 2048


def _mlp(emb, w1t, b1, w2t, b2):
    return pl.pallas_call(
        _mlp_body,
        grid=(B // BM,),
        in_specs=[
            pl.BlockSpec((BM, D), lambda i: (i, 0)),
            pl.BlockSpec((H1, D), lambda i: (0, 0)),
            pl.BlockSpec((H1,), lambda i: (0,)),
            pl.BlockSpec((H2, H1), lambda i: (0, 0)),
            pl.BlockSpec((H2,), lambda i: (0,)),
        ],
        out_specs=pl.BlockSpec((BM, H2), lambda i: (i, 0)),
        out_shape=jax.ShapeDtypeStruct((B, H2), jnp.float32),
    )(emb, w1t, b1, w2t, b2)


def kernel(class_labels, emb_table, W1, b1, W2, b2):
    idx = class_labels.astype(jnp.int32).reshape(NW, K, CH)
    emb = _sc_gather(emb_table, idx)
    return _mlp(emb, W1, b1, W2, b2)


# speedup vs baseline: 1.1365x; 1.0262x over previous
import functools

import jax
import jax.numpy as jnp
from jax import lax
from jax.experimental import pallas as pl
from jax.experimental.pallas import tpu as pltpu
from jax.experimental.pallas import tpu_sc as plsc

B = 16384
D = 128
H1 = 256
H2 = 512

NC = 2
NS = 16
NW = NC * NS
B_PER_W = B // NW
CH = 128
K = B_PER_W // CH

_sc_mesh = plsc.VectorSubcoreMesh(core_axis_name="c", subcore_axis_name="s")


@functools.partial(
    pl.kernel,
    out_type=jax.ShapeDtypeStruct((B, D), jnp.float32),
    mesh=_sc_mesh,
    scratch_types=[
        pltpu.VMEM((K, CH), jnp.int32),
        pltpu.VMEM((K, CH, D), jnp.float32),
        pltpu.SemaphoreType.DMA,
    ],
)
def _sc_gather(table_hbm, idx_hbm, out_hbm, idx_v, rows_v, sem):
    wid = lax.axis_index("s") * NC + lax.axis_index("c")
    pltpu.sync_copy(idx_hbm.at[wid], idx_v)
    copies = [
        pltpu.async_copy(table_hbm.at[idx_v.at[j]], rows_v.at[j], sem)
        for j in range(K)
    ]
    for j in range(K):
        copies[j].wait()
        pltpu.sync_copy(rows_v.at[j], out_hbm.at[pl.ds(wid * B_PER_W + j * CH, CH)])


def _mlp_body(emb_ref, w1_ref, b1_ref, w2_ref, b2_ref, out_ref):
    dn = (((1,), (1,)), ((), ()))
    h = lax.dot_general(emb_ref[...], w1_ref[...], dn,
                        preferred_element_type=jnp.float32)
    h = h + b1_ref[...]
    h = h * jax.nn.sigmoid(h)
    out_ref[...] = lax.dot_general(h, w2_ref[...], dn,
                                   preferred_element_type=jnp.float32) + b2_ref[...]


BM = 4096


def _mlp(emb, w1t, b1, w2t, b2):
    return pl.pallas_call(
        _mlp_body,
        grid=(B // BM,),
        in_specs=[
            pl.BlockSpec((BM, D), lambda i: (i, 0)),
            pl.BlockSpec((H1, D), lambda i: (0, 0)),
            pl.BlockSpec((H1,), lambda i: (0,)),
            pl.BlockSpec((H2, H1), lambda i: (0, 0)),
            pl.BlockSpec((H2,), lambda i: (0,)),
        ],
        out_specs=pl.BlockSpec((BM, H2), lambda i: (i, 0)),
        out_shape=jax.ShapeDtypeStruct((B, H2), jnp.float32),
    )(emb, w1t, b1, w2t, b2)


def kernel(class_labels, emb_table, W1, b1, W2, b2):
    idx = class_labels.astype(jnp.int32).reshape(NW, K, CH)
    emb = _sc_gather(emb_table, idx)
    return _mlp(emb, W1, b1, W2, b2)
